# Initial kernel scaffold; baseline (speedup 1.0000x reference)
import jax
import jax.numpy as jnp
from jax.experimental import pallas as pl


def kernel(x, edge_index, W1, a_src1, a_dst1, b1, W2, a_src2, a_dst2, b2, W3, a_src3, a_dst3, b3, fc_w, fc_b):
    raise NotImplementedError("write your pallas kernel here")


# speedup vs baseline: 74.5033x; 74.5033x over previous
import functools

import jax
import jax.numpy as jnp
from jax import lax
from jax.experimental import pallas as pl
from jax.experimental.pallas import tpu as pltpu
from jax.experimental.pallas import tpu_sc as plsc

N = 10000
E = 320000
HC = 128
NH = 8
CH = 16
HP = 16

BN = 2000
NSUB = 16
RA = 624
RL = N - 15 * RA
EB = 128
NBLK = E // EB
NW = 32
FULL_T = NBLK // NW
TAIL = NBLK - FULL_T * NW


def _head_stage(h, As_ref, Ad_ref, Bm_ref, h_out, asv_out, adv_out,
                acc0_out, den0_out):
  asv = jnp.dot(h, As_ref[...], preferred_element_type=jnp.float32, precision=jax.lax.Precision.HIGHEST)
  adv = jnp.dot(h, Ad_ref[...], preferred_element_type=jnp.float32, precision=jax.lax.Precision.HIGHEST)
  s0 = asv + adv
  w0 = jnp.exp(jnp.where(s0 > 0.0, s0, 0.2 * s0))
  h_out[...] = h
  asv_out[...] = asv
  adv_out[...] = adv
  acc0_out[...] = h * jnp.dot(w0, Bm_ref[...], preferred_element_type=jnp.float32, precision=jax.lax.Precision.HIGHEST)
  den0_out[...] = w0


def _tc_first_body(x_ref, W_ref, As_ref, Ad_ref, Bm_ref,
                   h_out, asv_out, adv_out, acc0_out, den0_out):
  h = jnp.dot(x_ref[...], W_ref[...], preferred_element_type=jnp.float32, precision=jax.lax.Precision.HIGHEST)
  _head_stage(h, As_ref, Ad_ref, Bm_ref, h_out, asv_out, adv_out,
              acc0_out, den0_out)


def _combine(acc0_ref, accO_ref, den0_ref, denO_ref, b_ref, Bm_ref):
  den = den0_ref[...] + denO_ref[0] + denO_ref[1]
  acc = acc0_ref[...] + accO_ref[0] + accO_ref[1]
  den128 = jnp.dot(den, Bm_ref[...], preferred_element_type=jnp.float32, precision=jax.lax.Precision.HIGHEST) + 1e-16
  g = acc / den128 + b_ref[...]
  return jnp.where(g > 0.0, g, jnp.exp(g) - 1.0)


def _tc_mid_body(acc0_ref, accO_ref, den0_ref, denO_ref, b_ref, W_ref,
                 As_ref, Ad_ref, Bm_ref,
                 h_out, asv_out, adv_out, acc0_out, den0_out):
  o = _combine(acc0_ref, accO_ref, den0_ref, denO_ref, b_ref, Bm_ref)
  h = jnp.dot(o, W_ref[...], preferred_element_type=jnp.float32, precision=jax.lax.Precision.HIGHEST)
  _head_stage(h, As_ref, Ad_ref, Bm_ref, h_out, asv_out, adv_out,
              acc0_out, den0_out)


def _tc_final_body(acc0_ref, accO_ref, den0_ref, denO_ref, b_ref, Bm_ref,
                   fcw_ref, fcb_ref, y_out):
  o = _combine(acc0_ref, accO_ref, den0_ref, denO_ref, b_ref, Bm_ref)
  y_out[...] = jnp.dot(o, fcw_ref[...], preferred_element_type=jnp.float32, precision=jax.lax.Precision.HIGHEST) + fcb_ref[0, 0]


def _full(shape):
  return pl.BlockSpec(shape, lambda i: tuple(0 for _ in shape))


def _rows(shape):
  return pl.BlockSpec(shape, lambda i: (i,) + tuple(0 for _ in shape[1:]))


def _rows3(shape):
  return pl.BlockSpec(shape, lambda i: (0, i) + tuple(0 for _ in shape[2:]))


_HEAD_OUT_SHAPES = [
    jax.ShapeDtypeStruct((N, HC), jnp.float32),
    jax.ShapeDtypeStruct((N, HP), jnp.float32),
    jax.ShapeDtypeStruct((N, HP), jnp.float32),
    jax.ShapeDtypeStruct((N, HC), jnp.float32),
    jax.ShapeDtypeStruct((N, HP), jnp.float32),
]
_HEAD_OUT_SPECS = [_rows((BN, HC)), _rows((BN, HP)), _rows((BN, HP)),
                   _rows((BN, HC)), _rows((BN, HP))]


def _make_tc_first(interpret=False):
  return pl.pallas_call(
      _tc_first_body,
      grid=(N // BN,),
      in_specs=[_rows((BN, HC)), _full((HC, HC)), _full((HC, HP)),
                _full((HC, HP)), _full((HP, HC))],
      out_specs=_HEAD_OUT_SPECS,
      out_shape=_HEAD_OUT_SHAPES,
      interpret=interpret,
  )


_AGG_IN_SPECS = [_rows((BN, HC)), _rows3((2, BN, HC)), _rows((BN, HP)),
                 _rows3((2, BN, HP)), _full((1, HC)), _full((HP, HC))]


def _make_tc_mid(interpret=False):
  return pl.pallas_call(
      _tc_mid_body,
      grid=(N // BN,),
      in_specs=_AGG_IN_SPECS[:5] + [_full((HC, HC)), _full((HC, HP)),
                                    _full((HC, HP)), _full((HP, HC))],
      out_specs=_HEAD_OUT_SPECS,
      out_shape=_HEAD_OUT_SHAPES,
      interpret=interpret,
  )


def _make_tc_final(interpret=False):
  return pl.pallas_call(
      _tc_final_body,
      grid=(N // BN,),
      in_specs=_AGG_IN_SPECS + [_full((HC, 1)), _full((1, 1))],
      out_specs=[_rows((BN, 1))],
      out_shape=[jax.ShapeDtypeStruct((N, 1), jnp.float32)],
      interpret=interpret,
  )


def _edge_body(src_hbm, dst_hbm, h_hbm, as_hbm, ad_hbm, zacc_hbm, zden_hbm,
               accO, denO, srcv, dstv, hrows, asr, adr, wv, acc_sh, den_sh,
               sem):
  c = lax.axis_index("c")
  s = lax.axis_index("s")
  wid = s * 2 + c
  r0 = s * RA

  @pl.when(s < NSUB - 1)
  def _zmain():
    pltpu.sync_copy(zacc_hbm.at[pl.ds(0, RA)], acc_sh.at[pl.ds(r0, RA)])
    pltpu.sync_copy(zden_hbm.at[pl.ds(0, RA)], den_sh.at[pl.ds(r0, RA)])

  @pl.when(s == NSUB - 1)
  def _ztail():
    pltpu.sync_copy(zacc_hbm, acc_sh.at[pl.ds(15 * RA, RL)])
    pltpu.sync_copy(zden_hbm, den_sh.at[pl.ds(15 * RA, RL)])

  plsc.subcore_barrier()

  def do_block(blk):
    base = blk * EB
    pltpu.sync_copy(src_hbm.at[pl.ds(base, EB)], srcv)
    pltpu.sync_copy(dst_hbm.at[pl.ds(base, EB)], dstv)
    cph = pltpu.async_copy(h_hbm.at[srcv], hrows, sem)
    cpa = pltpu.async_copy(as_hbm.at[srcv], asr, sem)
    cpd = pltpu.async_copy(ad_hbm.at[dstv], adr, sem)
    cph.wait()
    cpa.wait()
    cpd.wait()

    def ebody(e, carry):
      v = asr[e, :] + adr[e, :]
      w = jnp.exp(jnp.where(v > 0.0, v, 0.2 * v))
      wv[e, :] = w
      for hh in range(NH):
        ws = w[hh]
        hrows[e, pl.ds(hh * CH, CH)] = hrows[e, pl.ds(hh * CH, CH)] * ws
      return carry

    lax.fori_loop(0, EB, ebody, 0)
    pltpu.sync_copy(hrows, acc_sh.at[dstv], add=True)
    pltpu.sync_copy(wv, den_sh.at[dstv], add=True)

  def tbody(t, carry):
    do_block(wid + NW * t)
    return carry

  lax.fori_loop(0, FULL_T, tbody, 0)

  @pl.when(wid < TAIL)
  def _tail():
    do_block(FULL_T * NW + wid)

  plsc.subcore_barrier()

  @pl.when(s < NSUB - 1)
  def _wmain():
    pltpu.sync_copy(acc_sh.at[pl.ds(r0, RA)], accO.at[c, pl.ds(r0, RA)])
    pltpu.sync_copy(den_sh.at[pl.ds(r0, RA)], denO.at[c, pl.ds(r0, RA)])

  @pl.when(s == NSUB - 1)
  def _wtail():
    pltpu.sync_copy(acc_sh.at[pl.ds(15 * RA, RL)], accO.at[c, pl.ds(15 * RA, RL)])
    pltpu.sync_copy(den_sh.at[pl.ds(15 * RA, RL)], denO.at[c, pl.ds(15 * RA, RL)])


def _make_edge_kernel(interpret=False):
  mesh = plsc.VectorSubcoreMesh(core_axis_name="c", subcore_axis_name="s")
  return functools.partial(
      pl.kernel,
      out_type=[
          jax.ShapeDtypeStruct((2, N, HC), jnp.float32),
          jax.ShapeDtypeStruct((2, N, HP), jnp.float32),
      ],
      mesh=mesh,
      scratch_types=[
          pltpu.VMEM((EB,), jnp.int32),
          pltpu.VMEM((EB,), jnp.int32),
          pltpu.VMEM((EB, HC), jnp.float32),
          pltpu.VMEM((EB, HP), jnp.float32),
          pltpu.VMEM((EB, HP), jnp.float32),
          pltpu.VMEM((EB, HP), jnp.float32),
          pltpu.VMEM_SHARED((N, HC), jnp.float32),
          pltpu.VMEM_SHARED((N, HP), jnp.float32),
          pltpu.SemaphoreType.DMA,
      ],
      compiler_params=pltpu.CompilerParams(use_tc_tiling_on_sc=False),
      interpret=interpret,
  )(_edge_body)


def _att_mats(a_s, a_d):
  rows = jnp.arange(HC, dtype=jnp.int32)
  cols = rows // CH
  As = jnp.zeros((HC, HP), jnp.float32).at[rows, cols].set(a_s.reshape(-1))
  Ad = jnp.zeros((HC, HP), jnp.float32).at[rows, cols].set(a_d.reshape(-1))
  return As, Ad


def _build(interpret=False):
  tc_first = _make_tc_first(interpret)
  tc_mid = _make_tc_mid(interpret)
  tc_final = _make_tc_final(interpret)
  edge_k = _make_edge_kernel(interpret)

  def run(x, edge_index, W1, a_src1, a_dst1, b1, W2, a_src2, a_dst2, b2,
          W3, a_src3, a_dst3, b3, fc_w, fc_b):
    src = edge_index[0]
    dst = edge_index[1]
    zacc = jnp.zeros((RL, HC), jnp.float32)
    zden = jnp.zeros((RL, HP), jnp.float32)
    Bm = jnp.zeros((HP, HC), jnp.float32).at[
        jnp.arange(HC, dtype=jnp.int32) // CH,
        jnp.arange(HC, dtype=jnp.int32)].set(1.0)

    As1, Ad1 = _att_mats(a_src1, a_dst1)
    h, asv, adv, acc0, den0 = tc_first(x, W1, As1, Ad1, Bm)
    accO, denO = edge_k(src, dst, h, asv, adv, zacc, zden)

    As2, Ad2 = _att_mats(a_src2, a_dst2)
    h, asv, adv, acc0, den0 = tc_mid(acc0, accO, den0, denO,
                                     b1.reshape(1, HC), W2, As2, Ad2, Bm)
    accO, denO = edge_k(src, dst, h, asv, adv, zacc, zden)

    As3, Ad3 = _att_mats(a_src3, a_dst3)
    h, asv, adv, acc0, den0 = tc_mid(acc0, accO, den0, denO,
                                     b2.reshape(1, HC), W3, As3, Ad3, Bm)
    accO, denO = edge_k(src, dst, h, asv, adv, zacc, zden)

    (y,) = tc_final(acc0, accO, den0, denO, b3.reshape(1, HC), Bm,
                    fc_w, fc_b.reshape(1, 1))
    return y

  return run


_RUN = None


def kernel(x, edge_index, W1, a_src1, a_dst1, b1, W2, a_src2, a_dst2, b2,
           W3, a_src3, a_dst3, b3, fc_w, fc_b):
  global _RUN
  if _RUN is None:
    _RUN = jax.jit(_build())
  return _RUN(x, edge_index, W1, a_src1, a_dst1, b1, W2, a_src2, a_dst2, b2,
              W3, a_src3, a_dst3, b3, fc_w, fc_b)
